# baseline (speedup 1.0000x reference)
import functools
import jax
import jax.numpy as jnp
from jax import lax
from jax.experimental import pallas as pl
from jax.experimental.pallas import tpu as pltpu
from jax.experimental.pallas import tpu_sc as plsc

N_NODES = 10000
N_EDGES = 320000
NNZ_F = 320000
D_FEAT = 128
EMBED = 128
HIDDEN = 256
HOPS = 3
B_PAIRS = 4096

NC = 2
NS = 16
NW = NC * NS
L = 16

CH = 128
NCH = 80
EPT = NCH * CH
E_PAD = NW * EPT
RPT = 624
RTAIL = N_NODES - NS * RPT


@functools.partial(
    pl.kernel,
    out_type=jax.ShapeDtypeStruct((NC, N_NODES, EMBED), jnp.float32),
    mesh=plsc.VectorSubcoreMesh(core_axis_name="c", subcore_axis_name="s"),
    scratch_types=[
        pltpu.VMEM((NCH, CH), jnp.int32),
        pltpu.VMEM((CH,), jnp.int32),
        pltpu.VMEM((CH,), jnp.int32),
        pltpu.VMEM((CH * L,), jnp.float32),
        pltpu.VMEM((CH * L,), jnp.float32),
        pltpu.VMEM((CH, EMBED), jnp.float32),
        pltpu.VMEM((CH, EMBED), jnp.float32),
        pltpu.VMEM_SHARED((N_NODES, EMBED), jnp.float32),
        pltpu.SemaphoreType.DMA,
        pltpu.SemaphoreType.DMA,
        pltpu.SemaphoreType.DMA,
        pltpu.SemaphoreType.DMA,
        pltpu.SemaphoreType.DMA,
        pltpu.SemaphoreType.DMA,
        pltpu.SemaphoreType.DMA,
        pltpu.SemaphoreType.DMA,
    ],
)
def _sc_spmm(dst_hbm, src_hbm, vals_hbm, x_hbm, out_hbm,
             src_v, dstc0, dstc1, vexp0, vexp1, rows0, rows1, hsh,
             gsem0, gsem1, ssem0, ssem1, isem0, isem1, vsem0, vsem1):
    cid = lax.axis_index("c")
    sid = lax.axis_index("s")
    wid = sid * NC + cid
    dstc = (dstc0, dstc1)
    vexp = (vexp0, vexp1)
    rows = (rows0, rows1)
    gsem = (gsem0, gsem1)
    ssem = (ssem0, ssem1)
    isem = (isem0, isem1)
    vsem = (vsem0, vsem1)

    @pl.loop(0, CH)
    def _zero_rows(r):
        for g in range(EMBED // L):
            rows0[r, pl.ds(g * L, L)] = jnp.zeros((L,), jnp.float32)

    zbase = sid * RPT
    for k in range(RPT // CH):
        pltpu.sync_copy(rows0, hsh.at[pl.ds(zbase + k * CH, CH)])
    rem = RPT % CH
    if rem:
        pltpu.sync_copy(rows0.at[pl.ds(0, rem)],
                        hsh.at[pl.ds(zbase + (RPT // CH) * CH, rem)])

    @pl.when(sid == NS - 1)
    def _zero_tail():
        pltpu.sync_copy(rows0.at[pl.ds(0, RTAIL)],
                        hsh.at[pl.ds(NS * RPT, RTAIL)])

    plsc.subcore_barrier()

    pltpu.sync_copy(src_hbm.at[wid], src_v)

    pltpu.sync_copy(dst_hbm.at[wid, 0], dstc0)
    pltpu.async_copy(vals_hbm.at[wid, 0], vexp0, vsem0)
    pltpu.async_copy(dst_hbm.at[wid, 1], dstc1, isem1)
    pltpu.async_copy(x_hbm.at[dstc0], rows0, gsem0)

    @pl.loop(0, NCH // 2)
    def _pair(p):
        for b in range(2):
            nb = 1 - b
            c = p * 2 + b
            pltpu.make_async_copy(x_hbm.at[pl.ds(0, CH)], rows[b], gsem[b]).wait()

            @pl.when(c + 2 < NCH)
            def _pf_idx():
                pltpu.async_copy(dst_hbm.at[wid, c + 2], dstc[b], isem[b])

            @pl.when(c + 1 < NCH)
            def _launch_next():
                @pl.when(c >= 1)
                def _drain_prev_scatter():
                    pltpu.make_async_copy(
                        x_hbm.at[pl.ds(0, CH)], rows[nb], ssem[nb]).wait()
                pltpu.make_async_copy(dst_hbm.at[wid, 0], dstc[nb],
                                      isem[nb]).wait()
                pltpu.async_copy(x_hbm.at[dstc[nb]], rows[nb], gsem[nb])
                pltpu.async_copy(vals_hbm.at[wid, c + 1], vexp[nb], vsem[nb])

            pltpu.make_async_copy(vals_hbm.at[wid, 0], vexp[b], vsem[b]).wait()

            @plsc.parallel_loop(0, CH, 1, unroll=4)
            def _scale(r):
                vb = vexp[b][pl.ds(r * L, L)]
                for g in range(EMBED // L):
                    sl = pl.ds(g * L, L)
                    rows[b][r, sl] = rows[b][r, sl] * vb

            pltpu.async_copy(rows[b], hsh.at[src_v.at[c]], ssem[b], add=True)

    pltpu.make_async_copy(x_hbm.at[pl.ds(0, CH)], rows0, ssem0).wait()
    pltpu.make_async_copy(x_hbm.at[pl.ds(0, CH)], rows1, ssem1).wait()
    plsc.subcore_barrier()
    pltpu.sync_copy(hsh.at[pl.ds(zbase, RPT)],
                    out_hbm.at[cid, pl.ds(zbase, RPT)])

    @pl.when(sid == NS - 1)
    def _write_tail():
        pltpu.sync_copy(hsh.at[pl.ds(NS * RPT, RTAIL)],
                        out_hbm.at[cid, pl.ds(NS * RPT, RTAIL)])


def _spmm(row, col, vals, m, dense):
    gathered = vals[:, None] * dense[col]
    return jnp.zeros((m, dense.shape[1]), dense.dtype).at[row].add(gathered)


FCH = 128
FNCH = 157
FPT = FNCH * FCH
F_PAD = NS * FPT
XW = N_NODES * D_FEAT
XPT = XW // NS
XHALF = XW // NC

@functools.partial(
    pl.kernel,
    out_type=jax.ShapeDtypeStruct((XW,), jnp.float32),
    mesh=plsc.VectorSubcoreMesh(core_axis_name="c", subcore_axis_name="s"),
    scratch_types=[
        pltpu.VMEM((FNCH, FCH), jnp.int32),
        pltpu.VMEM((FNCH, FCH), jnp.float32),
        pltpu.VMEM((8000,), jnp.float32),
        pltpu.VMEM_SHARED((XW,), jnp.float32),
    ],
)
def _sc_densify(fidx_hbm, fvals_hbm, x_hbm, idx_v, vals_v, zero_v, xsh):
    cid = lax.axis_index("c")
    sid = lax.axis_index("s")

    @pl.loop(0, 500)
    def _zb(i):
        zero_v[pl.ds(i * L, L)] = jnp.zeros((L,), jnp.float32)

    for k in range(XPT // 8000):
        pltpu.sync_copy(zero_v, xsh.at[pl.ds(sid * XPT + k * 8000, 8000)])
    plsc.subcore_barrier()

    pltpu.sync_copy(fidx_hbm.at[sid], idx_v)
    pltpu.sync_copy(fvals_hbm.at[sid], vals_v)

    @pl.loop(0, FNCH)
    def _chunk(c):
        pltpu.sync_copy(vals_v.at[c], xsh.at[idx_v.at[c]], add=True)

    plsc.subcore_barrier()
    off = cid * XHALF + sid * (XHALF // NS)

    @pl.loop(0, (XHALF // NS) // 8000)
    def _wb(k):
        pltpu.sync_copy(xsh.at[pl.ds(off + k * 8000, 8000)], zero_v)
        pltpu.sync_copy(zero_v, x_hbm.at[pl.ds(off + k * 8000, 8000)])


@functools.partial(
    pl.kernel,
    out_type=jax.ShapeDtypeStruct((2 * B_PAIRS, EMBED), jnp.float32),
    mesh=plsc.VectorSubcoreMesh(core_axis_name="c", subcore_axis_name="s"),
    scratch_types=[
        pltpu.VMEM((2, 128), jnp.int32),
        pltpu.VMEM((128, EMBED), jnp.float32),
        pltpu.SemaphoreType.DMA,
    ],
)
def _sc_pair_gather(idx_hbm, enh_hbm, out_hbm, idx_v, rows_v, sem):
    cid = lax.axis_index("c")
    sid = lax.axis_index("s")
    wid = sid * NC + cid
    pltpu.sync_copy(idx_hbm.at[wid], idx_v)
    for k in range(2):
        pltpu.async_copy(enh_hbm.at[idx_v.at[k]], rows_v, sem).wait()
        pltpu.sync_copy(rows_v, out_hbm.at[pl.ds(wid * 256 + k * 128, 128)])


_TCB = 1000


def _hop_body(P_ref, Wt_ref, b_ref, attn_ref, h_ref, proj_ref, s_ref):
    hsum = P_ref[0] + P_ref[1]
    h_ref[...] = hsum
    proj = jnp.dot(hsum, Wt_ref[...], preferred_element_type=jnp.float32)
    proj = jnp.maximum(proj + b_ref[...], 0.0)
    proj_ref[...] = proj
    s_ref[...] = jnp.sum(proj * attn_ref[...], axis=1, keepdims=True)


def _tc_hop(P, Wt, bvec, attn):
    return pl.pallas_call(
        _hop_body,
        grid=(N_NODES // _TCB,),
        in_specs=[
            pl.BlockSpec((NC, _TCB, EMBED), lambda i: (0, i, 0)),
            pl.BlockSpec((EMBED, EMBED), lambda i: (0, 0)),
            pl.BlockSpec((1, EMBED), lambda i: (0, 0)),
            pl.BlockSpec((1, EMBED), lambda i: (0, 0)),
        ],
        out_specs=[
            pl.BlockSpec((_TCB, EMBED), lambda i: (i, 0)),
            pl.BlockSpec((_TCB, EMBED), lambda i: (i, 0)),
            pl.BlockSpec((_TCB, 1), lambda i: (i, 0)),
        ],
        out_shape=[
            jax.ShapeDtypeStruct((N_NODES, EMBED), jnp.float32),
            jax.ShapeDtypeStruct((N_NODES, EMBED), jnp.float32),
            jax.ShapeDtypeStruct((N_NODES, 1), jnp.float32),
        ],
    )(P, Wt, bvec.reshape(1, EMBED), attn.reshape(1, EMBED))


def _final_body(P_ref, Wt_ref, b_ref, attn_ref, p1_ref, p2_ref, s1_ref, s2_ref,
                enh_ref):
    hsum = P_ref[0] + P_ref[1]
    proj3 = jnp.dot(hsum, Wt_ref[...], preferred_element_type=jnp.float32)
    proj3 = jnp.maximum(proj3 + b_ref[...], 0.0)
    s3 = jnp.sum(proj3 * attn_ref[...], axis=1, keepdims=True)
    s1 = s1_ref[...]
    s2 = s2_ref[...]
    m = jnp.maximum(jnp.maximum(s1, s2), s3)
    e1 = jnp.exp(s1 - m)
    e2 = jnp.exp(s2 - m)
    e3 = jnp.exp(s3 - m)
    enh_ref[...] = (e1 * p1_ref[...] + e2 * p2_ref[...] + e3 * proj3) / (e1 + e2 + e3)


def _tc_final(P, Wt, bvec, attn, proj1, proj2, s1, s2):
    return pl.pallas_call(
        _final_body,
        grid=(N_NODES // _TCB,),
        in_specs=[
            pl.BlockSpec((NC, _TCB, EMBED), lambda i: (0, i, 0)),
            pl.BlockSpec((EMBED, EMBED), lambda i: (0, 0)),
            pl.BlockSpec((1, EMBED), lambda i: (0, 0)),
            pl.BlockSpec((1, EMBED), lambda i: (0, 0)),
            pl.BlockSpec((_TCB, EMBED), lambda i: (i, 0)),
            pl.BlockSpec((_TCB, EMBED), lambda i: (i, 0)),
            pl.BlockSpec((_TCB, 1), lambda i: (i, 0)),
            pl.BlockSpec((_TCB, 1), lambda i: (i, 0)),
        ],
        out_specs=pl.BlockSpec((_TCB, EMBED), lambda i: (i, 0)),
        out_shape=jax.ShapeDtypeStruct((N_NODES, EMBED), jnp.float32),
    )(P, Wt, bvec.reshape(1, EMBED), attn.reshape(1, EMBED), proj1, proj2,
      s1, s2)


def _dec_body(p1_ref, p2_ref, w1t_ref, b1_ref, w2t_ref, b2_ref,
              logits_ref, fused_ref):
    a = p1_ref[...]
    b = p2_ref[...]
    fused = jnp.concatenate([jnp.abs(a - b), a * b], axis=1)
    fused = jnp.where(fused > 0, fused, jnp.exp(fused) - 1.0)
    fused_ref[...] = fused
    h1 = jnp.dot(fused, w1t_ref[...], preferred_element_type=jnp.float32) + b1_ref[...]
    h1 = jnp.where(h1 > 0, h1, jnp.exp(h1) - 1.0)
    logits_ref[...] = (jnp.dot(h1, w2t_ref[...], preferred_element_type=jnp.float32)
                       + b2_ref[...])


def _decoder(p1, p2, dec_W1, dec_b1, dec_W2, dec_b2):
    B = 1024
    grid = (B_PAIRS // B,)
    return pl.pallas_call(
        _dec_body,
        grid=grid,
        in_specs=[
            pl.BlockSpec((B, EMBED), lambda i: (i, 0)),
            pl.BlockSpec((B, EMBED), lambda i: (i, 0)),
            pl.BlockSpec((2 * EMBED, HIDDEN), lambda i: (0, 0)),
            pl.BlockSpec((1, HIDDEN), lambda i: (0, 0)),
            pl.BlockSpec((HIDDEN, 1), lambda i: (0, 0)),
            pl.BlockSpec((1, 1), lambda i: (0, 0)),
        ],
        out_specs=[
            pl.BlockSpec((B, 1), lambda i: (i, 0)),
            pl.BlockSpec((B, 2 * EMBED), lambda i: (i, 0)),
        ],
        out_shape=[
            jax.ShapeDtypeStruct((B_PAIRS, 1), jnp.float32),
            jax.ShapeDtypeStruct((B_PAIRS, 2 * EMBED), jnp.float32),
        ],
    )(p1, p2, dec_W1.T, dec_b1.reshape(1, HIDDEN), dec_W2.T,
      dec_b2.reshape(1, 1))


@jax.jit
def kernel(feat_row, feat_col, feat_values, adj_src, adj_dst, adj_values, idx,
           W, b, attn_weights, dec_W1, dec_b1, dec_W2, dec_b2):
    fpad = F_PAD - NNZ_F
    flat_idx = jnp.pad((feat_row * D_FEAT + feat_col).astype(jnp.int32), (0, fpad))
    x = _sc_densify(flat_idx.reshape(NS, FNCH, FCH),
                    jnp.pad(feat_values, (0, fpad)).reshape(NS, FNCH, FCH))
    x = x.reshape(N_NODES, D_FEAT)
    pad = E_PAD - N_EDGES
    dst_p = jnp.pad(adj_dst.astype(jnp.int32), (0, pad))
    src_p = jnp.pad(adj_src.astype(jnp.int32), (0, pad))
    vals_p = jnp.pad(adj_values, (0, pad))
    dst3 = dst_p.reshape(NW, NCH, CH)
    src3 = src_p.reshape(NW, NCH, CH)
    vals3 = jnp.broadcast_to(
        vals_p.reshape(NW, NCH, CH, 1), (NW, NCH, CH, L)).reshape(NW, NCH, CH * L)
    P1 = _sc_spmm(dst3, src3, vals3, x)
    h1, proj1, s1 = _tc_hop(P1, W[0].T, b[0], attn_weights[0])
    P2 = _sc_spmm(dst3, src3, vals3, h1)
    h2, proj2, s2 = _tc_hop(P2, W[1].T, b[1], attn_weights[1])
    P3 = _sc_spmm(dst3, src3, vals3, h2)
    enhanced = _tc_final(P3, W[2].T, b[2], attn_weights[2], proj1, proj2, s1, s2)
    pidx = jnp.concatenate([idx[0], idx[1]]).astype(jnp.int32).reshape(NW, 2, 128)
    pairs = _sc_pair_gather(pidx, enhanced)
    feat_p1 = pairs[:B_PAIRS]
    feat_p2 = pairs[B_PAIRS:]
    logits, fused = _decoder(feat_p1, feat_p2, dec_W1, dec_b1, dec_W2, dec_b2)
    return (logits, fused)


# speedup vs baseline: 1.2933x; 1.2933x over previous
import functools
import jax
import jax.numpy as jnp
from jax import lax
from jax.experimental import pallas as pl
from jax.experimental.pallas import tpu as pltpu
from jax.experimental.pallas import tpu_sc as plsc

N_NODES = 10000
N_EDGES = 320000
NNZ_F = 320000
D_FEAT = 128
EMBED = 128
HIDDEN = 256
HOPS = 3
B_PAIRS = 4096

NC = 2
NS = 16
NW = NC * NS
L = 16

CH = 64
NBUF = 4
TOT_CH = 320
E_PAD = NS * TOT_CH * CH
SLOW_CID = 1
SLOW_NCH = 76
FAST_NCH = TOT_CH - SLOW_NCH
RPT = 624
RTAIL = N_NODES - NS * RPT


@functools.partial(
    pl.kernel,
    out_type=jax.ShapeDtypeStruct((NC, N_NODES, EMBED), jnp.float32),
    mesh=plsc.VectorSubcoreMesh(core_axis_name="c", subcore_axis_name="s"),
    scratch_types=(
        [pltpu.VMEM((1, CH), jnp.int32)] * NBUF
        + [pltpu.VMEM((CH,), jnp.int32)] * NBUF
        + [pltpu.VMEM((CH * L,), jnp.float32)] * NBUF
        + [pltpu.VMEM((CH, EMBED), jnp.float32)] * NBUF
        + [pltpu.VMEM_SHARED((N_NODES, EMBED), jnp.float32)]
        + [pltpu.SemaphoreType.DMA] * (5 * NBUF)
    ),
)
def _sc_spmm(dst_hbm, src_hbm, vals_hbm, x_hbm, out_hbm, *scr):
    cid = lax.axis_index("c")
    sid = lax.axis_index("s")
    srcc = scr[0:NBUF]
    dstc = scr[NBUF:2 * NBUF]
    vexp = scr[2 * NBUF:3 * NBUF]
    rows = scr[3 * NBUF:4 * NBUF]
    hsh = scr[4 * NBUF]
    sems = scr[4 * NBUF + 1:]
    gsem = sems[0:NBUF]
    ssem = sems[NBUF:2 * NBUF]
    isem = sems[2 * NBUF:3 * NBUF]
    vsem = sems[3 * NBUF:4 * NBUF]
    csem = sems[4 * NBUF:5 * NBUF]
    rows0 = rows[0]

    @pl.loop(0, CH)
    def _zero_rows(r):
        for g in range(EMBED // L):
            rows0[r, pl.ds(g * L, L)] = jnp.zeros((L,), jnp.float32)

    zbase = sid * RPT
    for k in range(RPT // CH):
        pltpu.sync_copy(rows0, hsh.at[pl.ds(zbase + k * CH, CH)])
    rem = RPT % CH
    if rem:
        pltpu.sync_copy(rows0.at[pl.ds(0, rem)],
                        hsh.at[pl.ds(zbase + (RPT // CH) * CH, rem)])

    @pl.when(sid == NS - 1)
    def _zero_tail():
        pltpu.sync_copy(rows0.at[pl.ds(0, RTAIL)],
                        hsh.at[pl.ds(NS * RPT, RTAIL)])

    plsc.subcore_barrier()

    def _run(base, nch):
        pltpu.sync_copy(dst_hbm.at[sid, base], dstc[0])
        for k in range(1, NBUF):
            pltpu.async_copy(dst_hbm.at[sid, base + k], dstc[k], isem[k])
        for k in range(NBUF - 1):
            pltpu.async_copy(vals_hbm.at[sid, base + k], vexp[k], vsem[k])
            pltpu.async_copy(src_hbm.at[sid, pl.ds(base + k, 1)], srcc[k],
                             csem[k])
        pltpu.async_copy(x_hbm.at[dstc[0]], rows[0], gsem[0])
        for k in range(1, NBUF - 1):
            pltpu.make_async_copy(dst_hbm.at[sid, 0], dstc[k], isem[k]).wait()
            pltpu.async_copy(x_hbm.at[dstc[k]], rows[k], gsem[k])

        @pl.loop(0, nch // NBUF)
        def _grp(p):
            for b in range(NBUF):
                bn = (b + NBUF - 1) % NBUF
                c = p * NBUF + b
                pltpu.make_async_copy(
                    x_hbm.at[pl.ds(0, CH)], rows[b], gsem[b]).wait()

                @pl.when(c + NBUF < nch)
                def _pf_idx():
                    pltpu.async_copy(dst_hbm.at[sid, base + c + NBUF], dstc[b],
                                     isem[b])

                @pl.when(c + NBUF - 1 < nch)
                def _launch_next():
                    @pl.when(c >= 1)
                    def _drain_prev_scatter():
                        pltpu.make_async_copy(
                            x_hbm.at[pl.ds(0, CH)], rows[bn], ssem[bn]).wait()
                    pltpu.make_async_copy(dst_hbm.at[sid, 0], dstc[bn],
                                          isem[bn]).wait()
                    pltpu.async_copy(x_hbm.at[dstc[bn]], rows[bn], gsem[bn])
                    pltpu.async_copy(vals_hbm.at[sid, base + c + NBUF - 1],
                                     vexp[bn], vsem[bn])
                    pltpu.async_copy(
                        src_hbm.at[sid, pl.ds(base + c + NBUF - 1, 1)],
                        srcc[bn], csem[bn])

                pltpu.make_async_copy(vals_hbm.at[sid, 0], vexp[b],
                                      vsem[b]).wait()

                @plsc.parallel_loop(0, CH, 1, unroll=4)
                def _scale(r):
                    vb = vexp[b][pl.ds(r * L, L)]
                    for g in range(EMBED // L):
                        sl = pl.ds(g * L, L)
                        rows[b][r, sl] = rows[b][r, sl] * vb

                pltpu.make_async_copy(src_hbm.at[sid, pl.ds(0, 1)], srcc[b],
                                      csem[b]).wait()
                pltpu.async_copy(rows[b], hsh.at[srcc[b].at[0]], ssem[b],
                                add=True)

        for k in range(NBUF):
            pltpu.make_async_copy(x_hbm.at[pl.ds(0, CH)], rows[k],
                                  ssem[k]).wait()

    @pl.when(cid == SLOW_CID)
    def _slow():
        _run(FAST_NCH, SLOW_NCH)

    @pl.when(cid != SLOW_CID)
    def _fast():
        _run(0, FAST_NCH)

    plsc.subcore_barrier()
    pltpu.sync_copy(hsh.at[pl.ds(zbase, RPT)],
                    out_hbm.at[cid, pl.ds(zbase, RPT)])

    @pl.when(sid == NS - 1)
    def _write_tail():
        pltpu.sync_copy(hsh.at[pl.ds(NS * RPT, RTAIL)],
                        out_hbm.at[cid, pl.ds(NS * RPT, RTAIL)])


def _spmm(row, col, vals, m, dense):
    gathered = vals[:, None] * dense[col]
    return jnp.zeros((m, dense.shape[1]), dense.dtype).at[row].add(gathered)


FCH = 128
FNCH = 157
FPT = FNCH * FCH
F_PAD = NS * FPT
XW = N_NODES * D_FEAT
XPT = XW // NS
XHALF = XW // NC

@functools.partial(
    pl.kernel,
    out_type=jax.ShapeDtypeStruct((XW,), jnp.float32),
    mesh=plsc.VectorSubcoreMesh(core_axis_name="c", subcore_axis_name="s"),
    scratch_types=[
        pltpu.VMEM((FNCH, FCH), jnp.int32),
        pltpu.VMEM((FNCH, FCH), jnp.float32),
        pltpu.VMEM((8000,), jnp.float32),
        pltpu.VMEM_SHARED((XW,), jnp.float32),
    ],
)
def _sc_densify(fidx_hbm, fvals_hbm, x_hbm, idx_v, vals_v, zero_v, xsh):
    cid = lax.axis_index("c")
    sid = lax.axis_index("s")

    @pl.loop(0, 500)
    def _zb(i):
        zero_v[pl.ds(i * L, L)] = jnp.zeros((L,), jnp.float32)

    for k in range(XPT // 8000):
        pltpu.sync_copy(zero_v, xsh.at[pl.ds(sid * XPT + k * 8000, 8000)])
    plsc.subcore_barrier()

    pltpu.sync_copy(fidx_hbm.at[sid], idx_v)
    pltpu.sync_copy(fvals_hbm.at[sid], vals_v)

    @pl.loop(0, FNCH)
    def _chunk(c):
        pltpu.sync_copy(vals_v.at[c], xsh.at[idx_v.at[c]], add=True)

    plsc.subcore_barrier()
    off = cid * XHALF + sid * (XHALF // NS)

    @pl.loop(0, (XHALF // NS) // 8000)
    def _wb(k):
        pltpu.sync_copy(xsh.at[pl.ds(off + k * 8000, 8000)], zero_v)
        pltpu.sync_copy(zero_v, x_hbm.at[pl.ds(off + k * 8000, 8000)])


@functools.partial(
    pl.kernel,
    out_type=jax.ShapeDtypeStruct((2 * B_PAIRS, EMBED), jnp.float32),
    mesh=plsc.VectorSubcoreMesh(core_axis_name="c", subcore_axis_name="s"),
    scratch_types=[
        pltpu.VMEM((2, 128), jnp.int32),
        pltpu.VMEM((128, EMBED), jnp.float32),
        pltpu.SemaphoreType.DMA,
    ],
)
def _sc_pair_gather(idx_hbm, enh_hbm, out_hbm, idx_v, rows_v, sem):
    cid = lax.axis_index("c")
    sid = lax.axis_index("s")
    wid = sid * NC + cid
    pltpu.sync_copy(idx_hbm.at[wid], idx_v)
    for k in range(2):
        pltpu.async_copy(enh_hbm.at[idx_v.at[k]], rows_v, sem).wait()
        pltpu.sync_copy(rows_v, out_hbm.at[pl.ds(wid * 256 + k * 128, 128)])


_TCB = 1000


def _hop_body(P_ref, Wt_ref, b_ref, attn_ref, h_ref, proj_ref, s_ref):
    hsum = P_ref[0] + P_ref[1]
    h_ref[...] = hsum
    proj = jnp.dot(hsum, Wt_ref[...], preferred_element_type=jnp.float32)
    proj = jnp.maximum(proj + b_ref[...], 0.0)
    proj_ref[...] = proj
    s_ref[...] = jnp.sum(proj * attn_ref[...], axis=1, keepdims=True)


def _tc_hop(P, Wt, bvec, attn):
    return pl.pallas_call(
        _hop_body,
        grid=(N_NODES // _TCB,),
        in_specs=[
            pl.BlockSpec((NC, _TCB, EMBED), lambda i: (0, i, 0)),
            pl.BlockSpec((EMBED, EMBED), lambda i: (0, 0)),
            pl.BlockSpec((1, EMBED), lambda i: (0, 0)),
            pl.BlockSpec((1, EMBED), lambda i: (0, 0)),
        ],
        out_specs=[
            pl.BlockSpec((_TCB, EMBED), lambda i: (i, 0)),
            pl.BlockSpec((_TCB, EMBED), lambda i: (i, 0)),
            pl.BlockSpec((_TCB, 1), lambda i: (i, 0)),
        ],
        out_shape=[
            jax.ShapeDtypeStruct((N_NODES, EMBED), jnp.float32),
            jax.ShapeDtypeStruct((N_NODES, EMBED), jnp.float32),
            jax.ShapeDtypeStruct((N_NODES, 1), jnp.float32),
        ],
    )(P, Wt, bvec.reshape(1, EMBED), attn.reshape(1, EMBED))


def _final_body(P_ref, Wt_ref, b_ref, attn_ref, p1_ref, p2_ref, s1_ref, s2_ref,
                enh_ref):
    hsum = P_ref[0] + P_ref[1]
    proj3 = jnp.dot(hsum, Wt_ref[...], preferred_element_type=jnp.float32)
    proj3 = jnp.maximum(proj3 + b_ref[...], 0.0)
    s3 = jnp.sum(proj3 * attn_ref[...], axis=1, keepdims=True)
    s1 = s1_ref[...]
    s2 = s2_ref[...]
    m = jnp.maximum(jnp.maximum(s1, s2), s3)
    e1 = jnp.exp(s1 - m)
    e2 = jnp.exp(s2 - m)
    e3 = jnp.exp(s3 - m)
    enh_ref[...] = (e1 * p1_ref[...] + e2 * p2_ref[...] + e3 * proj3) / (e1 + e2 + e3)


def _tc_final(P, Wt, bvec, attn, proj1, proj2, s1, s2):
    return pl.pallas_call(
        _final_body,
        grid=(N_NODES // _TCB,),
        in_specs=[
            pl.BlockSpec((NC, _TCB, EMBED), lambda i: (0, i, 0)),
            pl.BlockSpec((EMBED, EMBED), lambda i: (0, 0)),
            pl.BlockSpec((1, EMBED), lambda i: (0, 0)),
            pl.BlockSpec((1, EMBED), lambda i: (0, 0)),
            pl.BlockSpec((_TCB, EMBED), lambda i: (i, 0)),
            pl.BlockSpec((_TCB, EMBED), lambda i: (i, 0)),
            pl.BlockSpec((_TCB, 1), lambda i: (i, 0)),
            pl.BlockSpec((_TCB, 1), lambda i: (i, 0)),
        ],
        out_specs=pl.BlockSpec((_TCB, EMBED), lambda i: (i, 0)),
        out_shape=jax.ShapeDtypeStruct((N_NODES, EMBED), jnp.float32),
    )(P, Wt, bvec.reshape(1, EMBED), attn.reshape(1, EMBED), proj1, proj2,
      s1, s2)


def _dec_body(p1_ref, p2_ref, w1t_ref, b1_ref, w2t_ref, b2_ref,
              logits_ref, fused_ref):
    a = p1_ref[...]
    b = p2_ref[...]
    fused = jnp.concatenate([jnp.abs(a - b), a * b], axis=1)
    fused = jnp.where(fused > 0, fused, jnp.exp(fused) - 1.0)
    fused_ref[...] = fused
    h1 = jnp.dot(fused, w1t_ref[...], preferred_element_type=jnp.float32) + b1_ref[...]
    h1 = jnp.where(h1 > 0, h1, jnp.exp(h1) - 1.0)
    logits_ref[...] = (jnp.dot(h1, w2t_ref[...], preferred_element_type=jnp.float32)
                       + b2_ref[...])


def _decoder(p1, p2, dec_W1, dec_b1, dec_W2, dec_b2):
    B = 1024
    grid = (B_PAIRS // B,)
    return pl.pallas_call(
        _dec_body,
        grid=grid,
        in_specs=[
            pl.BlockSpec((B, EMBED), lambda i: (i, 0)),
            pl.BlockSpec((B, EMBED), lambda i: (i, 0)),
            pl.BlockSpec((2 * EMBED, HIDDEN), lambda i: (0, 0)),
            pl.BlockSpec((1, HIDDEN), lambda i: (0, 0)),
            pl.BlockSpec((HIDDEN, 1), lambda i: (0, 0)),
            pl.BlockSpec((1, 1), lambda i: (0, 0)),
        ],
        out_specs=[
            pl.BlockSpec((B, 1), lambda i: (i, 0)),
            pl.BlockSpec((B, 2 * EMBED), lambda i: (i, 0)),
        ],
        out_shape=[
            jax.ShapeDtypeStruct((B_PAIRS, 1), jnp.float32),
            jax.ShapeDtypeStruct((B_PAIRS, 2 * EMBED), jnp.float32),
        ],
    )(p1, p2, dec_W1.T, dec_b1.reshape(1, HIDDEN), dec_W2.T,
      dec_b2.reshape(1, 1))


@jax.jit
def kernel(feat_row, feat_col, feat_values, adj_src, adj_dst, adj_values, idx,
           W, b, attn_weights, dec_W1, dec_b1, dec_W2, dec_b2):
    fpad = F_PAD - NNZ_F
    flat_idx = jnp.pad((feat_row * D_FEAT + feat_col).astype(jnp.int32), (0, fpad))
    x = _sc_densify(flat_idx.reshape(NS, FNCH, FCH),
                    jnp.pad(feat_values, (0, fpad)).reshape(NS, FNCH, FCH))
    x = x.reshape(N_NODES, D_FEAT)
    pad = E_PAD - N_EDGES
    dst_p = jnp.pad(adj_dst.astype(jnp.int32), (0, pad))
    src_p = jnp.pad(adj_src.astype(jnp.int32), (0, pad))
    vals_p = jnp.pad(adj_values, (0, pad))
    dst3 = dst_p.reshape(NS, TOT_CH, CH)
    src3 = src_p.reshape(NS, TOT_CH, CH)
    vals3 = jnp.broadcast_to(
        vals_p.reshape(NS, TOT_CH, CH, 1),
        (NS, TOT_CH, CH, L)).reshape(NS, TOT_CH, CH * L)
    P1 = _sc_spmm(dst3, src3, vals3, x)
    h1, proj1, s1 = _tc_hop(P1, W[0].T, b[0], attn_weights[0])
    P2 = _sc_spmm(dst3, src3, vals3, h1)
    h2, proj2, s2 = _tc_hop(P2, W[1].T, b[1], attn_weights[1])
    P3 = _sc_spmm(dst3, src3, vals3, h2)
    enhanced = _tc_final(P3, W[2].T, b[2], attn_weights[2], proj1, proj2, s1, s2)
    pidx = jnp.concatenate([idx[0], idx[1]]).astype(jnp.int32).reshape(NW, 2, 128)
    pairs = _sc_pair_gather(pidx, enhanced)
    feat_p1 = pairs[:B_PAIRS]
    feat_p2 = pairs[B_PAIRS:]
    logits, fused = _decoder(feat_p1, feat_p2, dec_W1, dec_b1, dec_W2, dec_b2)
    return (logits, fused)
